# baseline (speedup 1.0000x reference)
import functools

import jax
import jax.numpy as jnp
from jax import lax
from jax.experimental import pallas as pl
from jax.experimental.pallas import tpu as pltpu
from jax.experimental.pallas import tpu_sc as plsc

L = 16
NC = 2
NS = 16
NW = NC * NS

TW = 128


def _wid():
    return lax.axis_index("s") * NC + lax.axis_index("c")


@functools.lru_cache(maxsize=None)
def _make_relayout(V, D):
    n_full = V // TW
    rem = V - n_full * TW
    n_iter = n_full // NW
    n_extra = n_full - n_iter * NW
    mesh = plsc.VectorSubcoreMesh(core_axis_name="c", subcore_axis_name="s")

    @functools.partial(
        pl.kernel,
        mesh=mesh,
        out_type=jax.ShapeDtypeStruct((V * D,), jnp.float32),
        scratch_types=[
            pltpu.VMEM((4, D, TW), jnp.float32),
            pltpu.VMEM((4, TW * D), jnp.float32),
            [pltpu.SemaphoreType.DMA] * 4,
            [pltpu.SemaphoreType.DMA] * 4,
        ],
        compiler_params=pltpu.CompilerParams(use_tc_tiling_on_sc=True, needs_layout_passes=False),
    )
    def k(tabT_hbm, tailT_hbm, lin_hbm, tile_v, out_v, isems, osems):
        w = _wid()
        lane = lax.broadcasted_iota(jnp.int32, (L,), 0)
        assert n_iter % 4 == 0
        UB = 4

        def fire_in(j, p):
            pltpu.async_copy(tabT_hbm.at[:, pl.ds(j * TW, TW)],
                             tile_v.at[p], isems[p])

        def wait_in(p):
            pltpu.make_async_copy(tabT_hbm.at[:, pl.ds(0, TW)],
                                  tile_v.at[p], isems[p]).wait()

        def fire_out(j, p):
            pltpu.async_copy(out_v.at[p],
                             lin_hbm.at[pl.ds(j * (TW * D), TW * D)], osems[p])

        def wait_out(p):
            pltpu.make_async_copy(out_v.at[p],
                                  lin_hbm.at[pl.ds(0, TW * D)], osems[p]).wait()

        def transpose_block(p):
            def b_body(bb, carry):
                b0 = bb * UB
                for u in range(UB):
                    col = jnp.full((L,), 0, jnp.int32) + (b0 + u)
                    for h in range(D // L):
                        v = plsc.load_gather(tile_v.at[p],
                                             [lane + (h * L), col])
                        out_v[p, pl.ds((b0 + u) * D + h * L, L)] = v
                return carry

            lax.fori_loop(0, TW // UB, b_body, 0)

        for p in range(3):
            fire_in(p * NW + w, p)

        def quad(t, carry):
            m0 = t * 4
            for u in range(4):
                m = m0 + u

                @pl.when(m + 3 < n_iter)
                def _():
                    fire_in((m + 3) * NW + w, (u + 3) % 4)

                wait_in(u)

                @pl.when(m >= 4)
                def _():
                    wait_out(u)

                transpose_block(u)
                fire_out(m * NW + w, u)
            return carry

        lax.fori_loop(0, n_iter // 4, quad, 0)
        for p in range(4):
            wait_out(p)

        @pl.when(w < n_extra)
        def _():
            j = n_iter * NW + w
            pltpu.sync_copy(tabT_hbm.at[:, pl.ds(j * TW, TW)], tile_v.at[0])
            transpose_block(0)
            pltpu.sync_copy(out_v.at[0],
                            lin_hbm.at[pl.ds(j * (TW * D), TW * D)])

        if rem:
            @pl.when(w == NW - 1)
            def _():
                pltpu.sync_copy(tailT_hbm, tile_v.at[1])
                transpose_block(1)
                pltpu.sync_copy(out_v.at[1],
                                lin_hbm.at[pl.ds((V - TW) * D, TW * D)])

    return k


@functools.lru_cache(maxsize=None)
def _make_gather(B, S, V, D):
    nj = B // TW
    assert nj == NW and S % 2 == 0
    mesh = plsc.VectorSubcoreMesh(core_axis_name="c", subcore_axis_name="s")

    @functools.partial(
        pl.kernel,
        mesh=mesh,
        out_type=(
            jax.ShapeDtypeStruct((S, D // 8, nj, 8, TW), jnp.float32),
            jax.ShapeDtypeStruct((S, B), jnp.float32),
        ),
        scratch_types=[
            pltpu.VMEM((S, TW), jnp.int32),
            pltpu.VMEM((4, TW, D), jnp.float32),
            pltpu.VMEM((4, D // 8, 8, TW), jnp.float32),
            pltpu.VMEM((B,), jnp.int32),
            pltpu.VMEM((B,), jnp.float32),
            [pltpu.SemaphoreType.DMA] * 4,
            [pltpu.SemaphoreType.DMA] * 4,
        ],
        compiler_params=pltpu.CompilerParams(
            use_tc_tiling_on_sc=False, needs_layout_passes=False),
    )
    def k(tab_hbm, idxT_hbm, res5_hbm, maskT_hbm,
          idx_v, rows_v, out_v, midx_v, msk_v, sems, osems):
        w = _wid()
        lane = lax.broadcasted_iota(jnp.int32, (L,), 0)
        n_quad = S // 4
        n_tail = S - n_quad * 4

        pltpu.sync_copy(idxT_hbm.at[:, pl.ds(w * TW, TW)], idx_v)

        def fire(s, p):
            return pltpu.async_copy(
                tab_hbm.at[idx_v.at[s]], rows_v.at[p], sems[p])

        def wait_rows(s, p):
            pltpu.make_async_copy(
                tab_hbm.at[idx_v.at[s]], rows_v.at[p], sems[p]).wait()

        def wait_outs(p):
            for i in range(D // 8):
                pltpu.make_async_copy(
                    out_v.at[p, i], res5_hbm.at[0, i, w], osems[p]).wait()

        def stage(s, p, first_round):
            wait_rows(s, p)
            if not first_round:
                wait_outs(p)

            def c_loop(ci, carry):
                for i in range(D // 8):
                    col = jnp.full((L,), 0, jnp.int32) + (i * 8 + ci)
                    for h in range(TW // L):
                        v = plsc.load_gather(
                            rows_v.at[p], [lane + (h * L), col])
                        out_v[p, i, ci, pl.ds(h * L, L)] = v
                return carry

            lax.fori_loop(0, 8, c_loop, 0)
            for i in range(D // 8):
                pltpu.async_copy(out_v.at[p, i], res5_hbm.at[s, i, w],
                                 osems[p])

        def stage_dyn(s, p, t):
            wait_rows(s, p)

            @pl.when(t > 0)
            def _():
                wait_outs(p)

            def c_loop(ci, carry):
                for i in range(D // 8):
                    col = jnp.full((L,), 0, jnp.int32) + (i * 8 + ci)
                    for h in range(TW // L):
                        v = plsc.load_gather(
                            rows_v.at[p], [lane + (h * L), col])
                        out_v[p, i, ci, pl.ds(h * L, L)] = v
                return carry

            lax.fori_loop(0, 8, c_loop, 0)
            for i in range(D // 8):
                pltpu.async_copy(out_v.at[p, i], res5_hbm.at[s, i, w],
                                 osems[p])

        for p in range(3):
            fire(p, p)

        def quad(t, carry):
            s0 = t * 4
            for u in range(4):
                s = s0 + u

                @pl.when(s + 3 < S)
                def _():
                    fire(s + 3, (u + 3) % 4)

                stage_dyn(s, u, t)
            return carry

        lax.fori_loop(0, n_quad, quad, 0)

        for u in range(n_tail):
            s = n_quad * 4 + u
            stage(s, u, first_round=False)
        for p in range(4):
            wait_outs(p)

        def mask_body(s):
            pltpu.sync_copy(idxT_hbm.at[s], midx_v)

            def cmp(i, c2):
                v = midx_v[pl.ds(i * L, L)]
                msk_v[pl.ds(i * L, L)] = jnp.where(
                    v == 0, jnp.float32(1.0), jnp.float32(0.0))
                return c2

            lax.fori_loop(0, B // L, cmp, 0)
            pltpu.sync_copy(msk_v, maskT_hbm.at[s])

        m_full = S // NW
        m_rem = S - m_full * NW

        def mask_outer(t, carry):
            mask_body(t * NW + w)
            return carry

        lax.fori_loop(0, m_full, mask_outer, 0)

        @pl.when(w < m_rem)
        def _():
            mask_body(m_full * NW + w)

    return k


def kernel(input, table):
    B, S = input.shape
    V, D = table.shape
    idxT = jnp.transpose(input)
    tabT = jnp.transpose(table)
    tailT = jnp.transpose(table[V - TW:])
    tab_lin = _make_relayout(V, D)(tabT, tailT).reshape(V, D)
    res5, maskT = _make_gather(B, S, V, D)(tab_lin, idxT)
    res = jnp.transpose(res5, (2, 4, 0, 1, 3)).reshape(B, S, D)
    mask = jnp.transpose(maskT)
    return res, mask


# speedup vs baseline: 1.1209x; 1.1209x over previous
import functools

import jax
import jax.numpy as jnp
from jax import lax
from jax.experimental import pallas as pl
from jax.experimental.pallas import tpu as pltpu
from jax.experimental.pallas import tpu_sc as plsc

L = 16
NC = 2
NS = 16
NW = NC * NS

TW = 128


def _wid():
    return lax.axis_index("s") * NC + lax.axis_index("c")


@functools.lru_cache(maxsize=None)
def _make_relayout(V, D):
    n_full = V // TW
    rem = V - n_full * TW
    n_iter = n_full // NW
    n_extra = n_full - n_iter * NW
    mesh = plsc.VectorSubcoreMesh(core_axis_name="c", subcore_axis_name="s")

    @functools.partial(
        pl.kernel,
        mesh=mesh,
        out_type=jax.ShapeDtypeStruct((V * D,), jnp.float32),
        scratch_types=[
            pltpu.VMEM((4, D, TW + 1), jnp.float32),
            pltpu.VMEM((4, TW * D), jnp.float32),
            [pltpu.SemaphoreType.DMA] * 4,
            [pltpu.SemaphoreType.DMA] * 4,
        ],
        compiler_params=pltpu.CompilerParams(use_tc_tiling_on_sc=True, needs_layout_passes=False),
    )
    def k(tabT_hbm, tailT_hbm, lin_hbm, tile_v, out_v, isems, osems):
        w = _wid()
        lane = lax.broadcasted_iota(jnp.int32, (L,), 0)
        assert n_iter % 4 == 0
        UB = 4

        def fire_in(j, p):
            for i in range(D // 8):
                pltpu.async_copy(
                    tabT_hbm.at[pl.ds(i * 8, 8), pl.ds(j * TW, TW)],
                    tile_v.at[p, pl.ds(i * 8, 8), pl.ds(0, TW)], isems[p])

        def wait_in(p):
            for i in range(D // 8):
                pltpu.make_async_copy(
                    tabT_hbm.at[pl.ds(i * 8, 8), pl.ds(0, TW)],
                    tile_v.at[p, pl.ds(i * 8, 8), pl.ds(0, TW)],
                    isems[p]).wait()

        def fire_out(j, p):
            pltpu.async_copy(out_v.at[p],
                             lin_hbm.at[pl.ds(j * (TW * D), TW * D)], osems[p])

        def wait_out(p):
            pltpu.make_async_copy(out_v.at[p],
                                  lin_hbm.at[pl.ds(0, TW * D)], osems[p]).wait()

        def transpose_block(p):
            def b_body(bb, carry):
                b0 = bb * UB
                for u in range(UB):
                    col = jnp.full((L,), 0, jnp.int32) + (b0 + u)
                    for h in range(D // L):
                        v = plsc.load_gather(tile_v.at[p],
                                             [lane + (h * L), col])
                        out_v[p, pl.ds((b0 + u) * D + h * L, L)] = v
                return carry

            lax.fori_loop(0, TW // UB, b_body, 0)

        for p in range(3):
            fire_in(p * NW + w, p)

        def quad(t, carry):
            m0 = t * 4
            for u in range(4):
                m = m0 + u

                @pl.when(m + 3 < n_iter)
                def _():
                    fire_in((m + 3) * NW + w, (u + 3) % 4)

                wait_in(u)

                @pl.when(m >= 4)
                def _():
                    wait_out(u)

                transpose_block(u)
                fire_out(m * NW + w, u)
            return carry

        lax.fori_loop(0, n_iter // 4, quad, 0)
        for p in range(4):
            wait_out(p)

        @pl.when(w < n_extra)
        def _():
            j = n_iter * NW + w
            fire_in(j, 0)
            wait_in(0)
            transpose_block(0)
            pltpu.sync_copy(out_v.at[0],
                            lin_hbm.at[pl.ds(j * (TW * D), TW * D)])

        if rem:
            @pl.when(w == NW - 1)
            def _():
                pltpu.sync_copy(tailT_hbm,
                                tile_v.at[1, :, pl.ds(0, TW)])
                transpose_block(1)
                pltpu.sync_copy(out_v.at[1],
                                lin_hbm.at[pl.ds((V - TW) * D, TW * D)])

    return k


@functools.lru_cache(maxsize=None)
def _make_gather(B, S, V, D):
    nj = B // TW
    assert nj == NW and S % 2 == 0
    mesh = plsc.VectorSubcoreMesh(core_axis_name="c", subcore_axis_name="s")

    @functools.partial(
        pl.kernel,
        mesh=mesh,
        out_type=(
            jax.ShapeDtypeStruct((S, D // 8, nj, 8, TW), jnp.float32),
            jax.ShapeDtypeStruct((S, B), jnp.float32),
        ),
        scratch_types=[
            pltpu.VMEM((S, TW), jnp.int32),
            pltpu.VMEM((4, TW, D), jnp.float32),
            pltpu.VMEM((4, D, TW + 1), jnp.float32),
            pltpu.VMEM((B,), jnp.int32),
            pltpu.VMEM((B,), jnp.float32),
            [pltpu.SemaphoreType.DMA] * 4,
            [pltpu.SemaphoreType.DMA] * 4,
        ],
        compiler_params=pltpu.CompilerParams(
            use_tc_tiling_on_sc=False, needs_layout_passes=False),
    )
    def k(tab_hbm, idxT_hbm, res5_hbm, maskT_hbm,
          idx_v, rows_v, out_v, midx_v, msk_v, sems, osems):
        w = _wid()
        lane = lax.broadcasted_iota(jnp.int32, (L,), 0)
        n_quad = S // 4
        n_tail = S - n_quad * 4

        pltpu.sync_copy(idxT_hbm.at[:, pl.ds(w * TW, TW)], idx_v)

        def fire(s, p):
            return pltpu.async_copy(
                tab_hbm.at[idx_v.at[s]], rows_v.at[p], sems[p])

        def wait_rows(s, p):
            pltpu.make_async_copy(
                tab_hbm.at[idx_v.at[s]], rows_v.at[p], sems[p]).wait()

        def out_src(p, i):
            return out_v.at[p, pl.ds(i * 8, 8), pl.ds(0, TW)]

        def wait_outs(p):
            for i in range(D // 8):
                pltpu.make_async_copy(
                    out_src(p, i), res5_hbm.at[0, i, w], osems[p]).wait()

        def transpose(p):
            UB = 4

            def b_body(bb, carry):
                for u in range(UB):
                    b = bb * UB + u
                    colv = jnp.full((L,), 0, jnp.int32) + b
                    for h in range(D // L):
                        v = rows_v[p, b, pl.ds(h * L, L)]
                        plsc.store_scatter(
                            out_v.at[p], [lane + (h * L), colv], v)
                return carry

            lax.fori_loop(0, TW // UB, b_body, 0)

        def stage(s, p, first_round):
            wait_rows(s, p)
            if not first_round:
                wait_outs(p)
            transpose(p)
            for i in range(D // 8):
                pltpu.async_copy(out_src(p, i), res5_hbm.at[s, i, w],
                                 osems[p])

        def stage_dyn(s, p, t):
            wait_rows(s, p)

            @pl.when(t > 0)
            def _():
                wait_outs(p)

            transpose(p)
            for i in range(D // 8):
                pltpu.async_copy(out_src(p, i), res5_hbm.at[s, i, w],
                                 osems[p])

        for p in range(3):
            fire(p, p)

        def quad(t, carry):
            s0 = t * 4
            for u in range(4):
                s = s0 + u

                @pl.when(s + 3 < S)
                def _():
                    fire(s + 3, (u + 3) % 4)

                stage_dyn(s, u, t)
            return carry

        lax.fori_loop(0, n_quad, quad, 0)

        for u in range(n_tail):
            s = n_quad * 4 + u
            stage(s, u, first_round=False)
        for p in range(4):
            wait_outs(p)

        def mask_body(s):
            pltpu.sync_copy(idxT_hbm.at[s], midx_v)

            def cmp(i, c2):
                v = midx_v[pl.ds(i * L, L)]
                msk_v[pl.ds(i * L, L)] = jnp.where(
                    v == 0, jnp.float32(1.0), jnp.float32(0.0))
                return c2

            lax.fori_loop(0, B // L, cmp, 0)
            pltpu.sync_copy(msk_v, maskT_hbm.at[s])

        m_full = S // NW
        m_rem = S - m_full * NW

        def mask_outer(t, carry):
            mask_body(t * NW + w)
            return carry

        lax.fori_loop(0, m_full, mask_outer, 0)

        @pl.when(w < m_rem)
        def _():
            mask_body(m_full * NW + w)

    return k


def kernel(input, table):
    B, S = input.shape
    V, D = table.shape
    idxT = jnp.transpose(input)
    tabT = jnp.transpose(table)
    tailT = jnp.transpose(table[V - TW:])
    tab_lin = _make_relayout(V, D)(tabT, tailT).reshape(V, D)
    res5, maskT = _make_gather(B, S, V, D)(tab_lin, idxT)
    res = jnp.transpose(res5, (2, 4, 0, 1, 3)).reshape(B, S, D)
    mask = jnp.transpose(maskT)
    return res, mask


# speedup vs baseline: 1.9990x; 1.7834x over previous
import functools

import jax
import jax.numpy as jnp
from jax import lax
from jax.experimental import pallas as pl
from jax.experimental.pallas import tpu as pltpu
from jax.experimental.pallas import tpu_sc as plsc

L = 16
NC = 2
NS = 16
NW = NC * NS

TW = 128


def _wid():
    return lax.axis_index("s") * NC + lax.axis_index("c")


@functools.lru_cache(maxsize=None)
def _make_relayout(V, D):
    n_full = V // TW
    rem = V - n_full * TW
    n_iter = n_full // NW
    n_extra = n_full - n_iter * NW
    mesh = plsc.VectorSubcoreMesh(core_axis_name="c", subcore_axis_name="s")

    @functools.partial(
        pl.kernel,
        mesh=mesh,
        out_type=jax.ShapeDtypeStruct((V * D,), jnp.float32),
        scratch_types=[
            pltpu.VMEM((4, D, TW + 1), jnp.float32),
            pltpu.VMEM((4, TW * D), jnp.float32),
            [pltpu.SemaphoreType.DMA] * 4,
            [pltpu.SemaphoreType.DMA] * 4,
        ],
        compiler_params=pltpu.CompilerParams(use_tc_tiling_on_sc=True, needs_layout_passes=False),
    )
    def k(tabT_hbm, tailT_hbm, lin_hbm, tile_v, out_v, isems, osems):
        w = _wid()
        lane = lax.broadcasted_iota(jnp.int32, (L,), 0)
        assert n_iter % 4 == 0
        UB = 4

        def fire_in(j, p):
            for i in range(D // 8):
                pltpu.async_copy(
                    tabT_hbm.at[pl.ds(i * 8, 8), pl.ds(j * TW, TW)],
                    tile_v.at[p, pl.ds(i * 8, 8), pl.ds(0, TW)], isems[p])

        def wait_in(p):
            for i in range(D // 8):
                pltpu.make_async_copy(
                    tabT_hbm.at[pl.ds(i * 8, 8), pl.ds(0, TW)],
                    tile_v.at[p, pl.ds(i * 8, 8), pl.ds(0, TW)],
                    isems[p]).wait()

        def fire_out(j, p):
            pltpu.async_copy(out_v.at[p],
                             lin_hbm.at[pl.ds(j * (TW * D), TW * D)], osems[p])

        def wait_out(p):
            pltpu.make_async_copy(out_v.at[p],
                                  lin_hbm.at[pl.ds(0, TW * D)], osems[p]).wait()

        def transpose_block(p):
            @plsc.parallel_loop(0, TW, unroll=8)
            def _(b):
                col = jnp.full((L,), 0, jnp.int32) + b
                for h in range(D // L):
                    v = plsc.load_gather(tile_v.at[p],
                                         [lane + (h * L), col])
                    out_v[p, pl.ds(b * D + h * L, L)] = v

        for p in range(3):
            fire_in(p * NW + w, p)

        def quad(t, carry):
            m0 = t * 4
            for u in range(4):
                m = m0 + u

                @pl.when(m + 3 < n_iter)
                def _():
                    fire_in((m + 3) * NW + w, (u + 3) % 4)

                wait_in(u)

                @pl.when(m >= 4)
                def _():
                    wait_out(u)

                transpose_block(u)
                fire_out(m * NW + w, u)
            return carry

        lax.fori_loop(0, n_iter // 4, quad, 0)
        for p in range(4):
            wait_out(p)

        @pl.when(w < n_extra)
        def _():
            j = n_iter * NW + w
            fire_in(j, 0)
            wait_in(0)
            transpose_block(0)
            pltpu.sync_copy(out_v.at[0],
                            lin_hbm.at[pl.ds(j * (TW * D), TW * D)])

        if rem:
            @pl.when(w == NW - 1)
            def _():
                pltpu.sync_copy(tailT_hbm,
                                tile_v.at[1, :, pl.ds(0, TW)])
                transpose_block(1)
                pltpu.sync_copy(out_v.at[1],
                                lin_hbm.at[pl.ds((V - TW) * D, TW * D)])

    return k


@functools.lru_cache(maxsize=None)
def _make_gather(B, S, V, D):
    nj = B // TW
    assert nj == NW and S % 2 == 0
    mesh = plsc.VectorSubcoreMesh(core_axis_name="c", subcore_axis_name="s")

    @functools.partial(
        pl.kernel,
        mesh=mesh,
        out_type=(
            jax.ShapeDtypeStruct((S, D // 8, nj, 8, TW), jnp.float32),
            jax.ShapeDtypeStruct((S, B), jnp.float32),
        ),
        scratch_types=[
            pltpu.VMEM((S, TW), jnp.int32),
            pltpu.VMEM((4, TW, D), jnp.float32),
            pltpu.VMEM((4, D, TW + 1), jnp.float32),
            pltpu.VMEM((B,), jnp.int32),
            pltpu.VMEM((B,), jnp.float32),
            [pltpu.SemaphoreType.DMA] * 4,
            [pltpu.SemaphoreType.DMA] * 4,
        ],
        compiler_params=pltpu.CompilerParams(
            use_tc_tiling_on_sc=False, needs_layout_passes=False),
    )
    def k(tab_hbm, idxT_hbm, res5_hbm, maskT_hbm,
          idx_v, rows_v, out_v, midx_v, msk_v, sems, osems):
        w = _wid()
        lane = lax.broadcasted_iota(jnp.int32, (L,), 0)
        n_quad = S // 4
        n_tail = S - n_quad * 4

        pltpu.sync_copy(idxT_hbm.at[:, pl.ds(w * TW, TW)], idx_v)

        def fire(s, p):
            return pltpu.async_copy(
                tab_hbm.at[idx_v.at[s]], rows_v.at[p], sems[p])

        def wait_rows(s, p):
            pltpu.make_async_copy(
                tab_hbm.at[idx_v.at[s]], rows_v.at[p], sems[p]).wait()

        def out_src(p, i):
            return out_v.at[p, pl.ds(i * 8, 8), pl.ds(0, TW)]

        def wait_outs(p):
            for i in range(D // 8):
                pltpu.make_async_copy(
                    out_src(p, i), res5_hbm.at[0, i, w], osems[p]).wait()

        def transpose(p):
            @plsc.parallel_loop(0, TW, unroll=8)
            def _(b):
                colv = jnp.full((L,), 0, jnp.int32) + b
                for h in range(D // L):
                    v = rows_v[p, b, pl.ds(h * L, L)]
                    plsc.store_scatter(
                        out_v.at[p], [lane + (h * L), colv], v)

        def stage(s, p, first_round):
            wait_rows(s, p)
            if not first_round:
                wait_outs(p)
            transpose(p)
            for i in range(D // 8):
                pltpu.async_copy(out_src(p, i), res5_hbm.at[s, i, w],
                                 osems[p])

        def stage_dyn(s, p, t):
            wait_rows(s, p)

            @pl.when(t > 0)
            def _():
                wait_outs(p)

            transpose(p)
            for i in range(D // 8):
                pltpu.async_copy(out_src(p, i), res5_hbm.at[s, i, w],
                                 osems[p])

        for p in range(3):
            fire(p, p)

        def quad(t, carry):
            s0 = t * 4
            for u in range(4):
                s = s0 + u

                @pl.when(s + 3 < S)
                def _():
                    fire(s + 3, (u + 3) % 4)

                stage_dyn(s, u, t)
            return carry

        lax.fori_loop(0, n_quad, quad, 0)

        for u in range(n_tail):
            s = n_quad * 4 + u
            stage(s, u, first_round=False)
        for p in range(4):
            wait_outs(p)

        def mask_body(s):
            pltpu.sync_copy(idxT_hbm.at[s], midx_v)

            @plsc.parallel_loop(0, B // L, unroll=8)
            def _(i):
                v = midx_v[pl.ds(i * L, L)]
                msk_v[pl.ds(i * L, L)] = jnp.where(
                    v == 0, jnp.float32(1.0), jnp.float32(0.0))
            pltpu.sync_copy(msk_v, maskT_hbm.at[s])

        m_full = S // NW
        m_rem = S - m_full * NW

        def mask_outer(t, carry):
            mask_body(t * NW + w)
            return carry

        lax.fori_loop(0, m_full, mask_outer, 0)

        @pl.when(w < m_rem)
        def _():
            mask_body(m_full * NW + w)

    return k


def kernel(input, table):
    B, S = input.shape
    V, D = table.shape
    idxT = jnp.transpose(input)
    tabT = jnp.transpose(table)
    tailT = jnp.transpose(table[V - TW:])
    tab_lin = _make_relayout(V, D)(tabT, tailT).reshape(V, D)
    res5, maskT = _make_gather(B, S, V, D)(tab_lin, idxT)
    res = jnp.transpose(res5, (2, 4, 0, 1, 3)).reshape(B, S, D)
    mask = jnp.transpose(maskT)
    return res, mask


# speedup vs baseline: 1.9991x; 1.0001x over previous
import functools

import jax
import jax.numpy as jnp
from jax import lax
from jax.experimental import pallas as pl
from jax.experimental.pallas import tpu as pltpu
from jax.experimental.pallas import tpu_sc as plsc

L = 16
NC = 2
NS = 16
NW = NC * NS

TW = 128


def _wid():
    return lax.axis_index("s") * NC + lax.axis_index("c")


@functools.lru_cache(maxsize=None)
def _make_relayout(V, D):
    n_full = V // TW
    rem = V - n_full * TW
    n_iter = n_full // NW
    n_extra = n_full - n_iter * NW
    mesh = plsc.VectorSubcoreMesh(core_axis_name="c", subcore_axis_name="s")

    @functools.partial(
        pl.kernel,
        mesh=mesh,
        out_type=jax.ShapeDtypeStruct((V * D,), jnp.float32),
        scratch_types=[
            pltpu.VMEM((4, D, TW + 8), jnp.float32),
            pltpu.VMEM((4, TW * D), jnp.float32),
            [pltpu.SemaphoreType.DMA] * 4,
            [pltpu.SemaphoreType.DMA] * 4,
        ],
        compiler_params=pltpu.CompilerParams(use_tc_tiling_on_sc=True, needs_layout_passes=False),
    )
    def k(tabT_hbm, tailT_hbm, lin_hbm, tile_v, out_v, isems, osems):
        w = _wid()
        lane = lax.broadcasted_iota(jnp.int32, (L,), 0)
        assert n_iter % 4 == 0
        UB = 4

        def fire_in(j, p):
            for i in range(D // 8):
                pltpu.async_copy(
                    tabT_hbm.at[pl.ds(i * 8, 8), pl.ds(j * TW, TW)],
                    tile_v.at[p, pl.ds(i * 8, 8), pl.ds(0, TW)], isems[p])

        def wait_in(p):
            for i in range(D // 8):
                pltpu.make_async_copy(
                    tabT_hbm.at[pl.ds(i * 8, 8), pl.ds(0, TW)],
                    tile_v.at[p, pl.ds(i * 8, 8), pl.ds(0, TW)],
                    isems[p]).wait()

        def fire_out(j, p):
            pltpu.async_copy(out_v.at[p],
                             lin_hbm.at[pl.ds(j * (TW * D), TW * D)], osems[p])

        def wait_out(p):
            pltpu.make_async_copy(out_v.at[p],
                                  lin_hbm.at[pl.ds(0, TW * D)], osems[p]).wait()

        def transpose_block(p):
            @plsc.parallel_loop(0, TW, unroll=8)
            def _(b):
                col = jnp.full((L,), 0, jnp.int32) + b
                for h in range(D // L):
                    v = plsc.load_gather(tile_v.at[p],
                                         [lane + (h * L), col])
                    out_v[p, pl.ds(b * D + h * L, L)] = v

        for p in range(3):
            fire_in(p * NW + w, p)

        def quad(t, carry):
            m0 = t * 4
            for u in range(4):
                m = m0 + u

                @pl.when(m + 3 < n_iter)
                def _():
                    fire_in((m + 3) * NW + w, (u + 3) % 4)

                wait_in(u)

                @pl.when(m >= 4)
                def _():
                    wait_out(u)

                transpose_block(u)
                fire_out(m * NW + w, u)
            return carry

        lax.fori_loop(0, n_iter // 4, quad, 0)
        for p in range(4):
            wait_out(p)

        @pl.when(w < n_extra)
        def _():
            j = n_iter * NW + w
            fire_in(j, 0)
            wait_in(0)
            transpose_block(0)
            pltpu.sync_copy(out_v.at[0],
                            lin_hbm.at[pl.ds(j * (TW * D), TW * D)])

        if rem:
            @pl.when(w == NW - 1)
            def _():
                pltpu.sync_copy(tailT_hbm,
                                tile_v.at[1, :, pl.ds(0, TW)])
                transpose_block(1)
                pltpu.sync_copy(out_v.at[1],
                                lin_hbm.at[pl.ds((V - TW) * D, TW * D)])

    return k


@functools.lru_cache(maxsize=None)
def _make_gather(B, S, V, D):
    nj = B // TW
    assert nj == NW and S % 2 == 0
    mesh = plsc.VectorSubcoreMesh(core_axis_name="c", subcore_axis_name="s")

    @functools.partial(
        pl.kernel,
        mesh=mesh,
        out_type=(
            jax.ShapeDtypeStruct((S, D // 8, nj, 8, TW), jnp.float32),
            jax.ShapeDtypeStruct((S, B), jnp.float32),
        ),
        scratch_types=[
            pltpu.VMEM((S, TW), jnp.int32),
            pltpu.VMEM((4, TW, D), jnp.float32),
            pltpu.VMEM((4, D, TW + 1), jnp.float32),
            pltpu.VMEM((B,), jnp.int32),
            pltpu.VMEM((B,), jnp.float32),
            [pltpu.SemaphoreType.DMA] * 4,
            [pltpu.SemaphoreType.DMA] * 4,
        ],
        compiler_params=pltpu.CompilerParams(
            use_tc_tiling_on_sc=False, needs_layout_passes=False),
    )
    def k(tab_hbm, idxT_hbm, res5_hbm, maskT_hbm,
          idx_v, rows_v, out_v, midx_v, msk_v, sems, osems):
        w = _wid()
        lane = lax.broadcasted_iota(jnp.int32, (L,), 0)
        n_quad = S // 4
        n_tail = S - n_quad * 4

        pltpu.sync_copy(idxT_hbm.at[:, pl.ds(w * TW, TW)], idx_v)

        def fire(s, p):
            return pltpu.async_copy(
                tab_hbm.at[idx_v.at[s]], rows_v.at[p], sems[p])

        def wait_rows(s, p):
            pltpu.make_async_copy(
                tab_hbm.at[idx_v.at[s]], rows_v.at[p], sems[p]).wait()

        def out_src(p, i):
            return out_v.at[p, pl.ds(i * 8, 8), pl.ds(0, TW)]

        def wait_outs(p):
            for i in range(D // 8):
                pltpu.make_async_copy(
                    out_src(p, i), res5_hbm.at[0, i, w], osems[p]).wait()

        def transpose(p):
            @plsc.parallel_loop(0, TW, unroll=8)
            def _(b):
                colv = jnp.full((L,), 0, jnp.int32) + b
                for h in range(D // L):
                    v = rows_v[p, b, pl.ds(h * L, L)]
                    plsc.store_scatter(
                        out_v.at[p], [lane + (h * L), colv], v)

        def stage(s, p, first_round):
            wait_rows(s, p)
            if not first_round:
                wait_outs(p)
            transpose(p)
            for i in range(D // 8):
                pltpu.async_copy(out_src(p, i), res5_hbm.at[s, i, w],
                                 osems[p])

        def stage_dyn(s, p, t):
            wait_rows(s, p)

            @pl.when(t > 0)
            def _():
                wait_outs(p)

            transpose(p)
            for i in range(D // 8):
                pltpu.async_copy(out_src(p, i), res5_hbm.at[s, i, w],
                                 osems[p])

        for p in range(3):
            fire(p, p)

        def quad(t, carry):
            s0 = t * 4
            for u in range(4):
                s = s0 + u

                @pl.when(s + 3 < S)
                def _():
                    fire(s + 3, (u + 3) % 4)

                stage_dyn(s, u, t)
            return carry

        lax.fori_loop(0, n_quad, quad, 0)

        for u in range(n_tail):
            s = n_quad * 4 + u
            stage(s, u, first_round=False)
        for p in range(4):
            wait_outs(p)

        def mask_body(s):
            pltpu.sync_copy(idxT_hbm.at[s], midx_v)

            @plsc.parallel_loop(0, B // L, unroll=8)
            def _(i):
                v = midx_v[pl.ds(i * L, L)]
                msk_v[pl.ds(i * L, L)] = jnp.where(
                    v == 0, jnp.float32(1.0), jnp.float32(0.0))
            pltpu.sync_copy(msk_v, maskT_hbm.at[s])

        m_full = S // NW
        m_rem = S - m_full * NW

        def mask_outer(t, carry):
            mask_body(t * NW + w)
            return carry

        lax.fori_loop(0, m_full, mask_outer, 0)

        @pl.when(w < m_rem)
        def _():
            mask_body(m_full * NW + w)

    return k


def kernel(input, table):
    B, S = input.shape
    V, D = table.shape
    idxT = jnp.transpose(input)
    tabT = jnp.transpose(table)
    tailT = jnp.transpose(table[V - TW:])
    tab_lin = _make_relayout(V, D)(tabT, tailT).reshape(V, D)
    res5, maskT = _make_gather(B, S, V, D)(tab_lin, idxT)
    res = jnp.transpose(res5, (2, 4, 0, 1, 3)).reshape(B, S, D)
    mask = jnp.transpose(maskT)
    return res, mask
